# baseline (speedup 1.0000x reference)
import functools

import jax
import jax.numpy as jnp
from jax import lax
from jax.experimental import pallas as pl
from jax.experimental.pallas import tpu as pltpu
from jax.experimental.pallas import tpu_sc as plsc

N_NODES = 50000
N_EDGES = 800000

SC_CORES = 2
SC_SUBCORES = 16
SC_TILES = SC_CORES * SC_SUBCORES

CHUNK = 128
PER_TILE_CHUNKS = 200
EP = SC_TILES * PER_TILE_CHUNKS * CHUNK
IDX_ROWS = 8
OUTER = PER_TILE_CHUNKS // IDX_ROWS
STEP = IDX_ROWS * CHUNK
EROWS = EP // CHUNK

NPAD = 50048
ZROWS = NPAD // SC_SUBCORES

_SELU_ALPHA = 1.6732632423543772
_SELU_SCALE = 1.0507009873554805

EB = 2048
NB = 2944


def _selu(x):
    return _SELU_SCALE * jnp.where(x > 0, x, _SELU_ALPHA * (jnp.exp(x) - 1.0))


def _sc_gather_body(tab, recv2, send2, nr, ns, ir, is_, rr, rs, s1, s2):
    wid = lax.axis_index("s") * SC_CORES + lax.axis_index("c")

    def step(t, carry):
        row0 = wid * PER_TILE_CHUNKS + t * IDX_ROWS
        e0 = row0 * CHUNK
        pltpu.sync_copy(recv2.at[pl.ds(row0, IDX_ROWS)], ir)
        pltpu.sync_copy(send2.at[pl.ds(row0, IDX_ROWS)], is_)
        hs = []
        for j in range(IDX_ROWS):
            hs.append(pltpu.async_copy(
                tab.at[ir.at[j]], rr.at[pl.ds(j * CHUNK, CHUNK)], s1))
            hs.append(pltpu.async_copy(
                tab.at[is_.at[j]], rs.at[pl.ds(j * CHUNK, CHUNK)], s2))
        for h in hs:
            h.wait()
        pltpu.sync_copy(rr, nr.at[pl.ds(e0, STEP)])
        pltpu.sync_copy(rs, ns.at[pl.ds(e0, STEP)])
        return carry

    lax.fori_loop(0, OUTER, step, 0)


@functools.cache
def _sc_gather():
    return pl.kernel(
        _sc_gather_body,
        out_type=(jax.ShapeDtypeStruct((EP, 8), jnp.float32),
                  jax.ShapeDtypeStruct((EP, 8), jnp.float32)),
        mesh=plsc.VectorSubcoreMesh(core_axis_name="c", subcore_axis_name="s"),
        scratch_types=[
            pltpu.VMEM((IDX_ROWS, CHUNK), jnp.int32),
            pltpu.VMEM((IDX_ROWS, CHUNK), jnp.int32),
            pltpu.VMEM((STEP, 8), jnp.float32),
            pltpu.VMEM((STEP, 8), jnp.float32),
            pltpu.SemaphoreType.DMA,
            pltpu.SemaphoreType.DMA,
        ],
        compiler_params=pltpu.CompilerParams(use_tc_tiling_on_sc=False),
    )


def _sc_scatter_body(h1, recv2, zrows, out, shared, ib, rv, zb):
    cid = lax.axis_index("c")
    sid = lax.axis_index("s")
    wid = sid * SC_CORES + cid
    r0 = sid * ZROWS

    pltpu.sync_copy(zrows, zb)
    pltpu.sync_copy(zb, shared.at[pl.ds(r0, ZROWS)])
    plsc.subcore_barrier()

    def step(t, carry):
        row0 = wid * PER_TILE_CHUNKS + t * IDX_ROWS
        e0 = row0 * CHUNK
        pltpu.sync_copy(recv2.at[pl.ds(row0, IDX_ROWS)], ib)
        pltpu.sync_copy(h1.at[pl.ds(e0, STEP)], rv)
        for j in range(IDX_ROWS):
            pltpu.sync_copy(rv.at[pl.ds(j * CHUNK, CHUNK)],
                            shared.at[ib.at[j]], add=True)
        return carry

    lax.fori_loop(0, OUTER, step, 0)
    plsc.subcore_barrier()
    pltpu.sync_copy(shared.at[pl.ds(r0, ZROWS)], zb)
    pltpu.sync_copy(zb, out.at[cid, pl.ds(r0, ZROWS)])


@functools.cache
def _sc_scatter():
    return pl.kernel(
        _sc_scatter_body,
        out_type=jax.ShapeDtypeStruct((SC_CORES, NPAD, 8), jnp.float32),
        mesh=plsc.VectorSubcoreMesh(core_axis_name="c", subcore_axis_name="s"),
        scratch_types=[
            pltpu.VMEM_SHARED((NPAD, 8), jnp.float32),
            pltpu.VMEM((IDX_ROWS, CHUNK), jnp.int32),
            pltpu.VMEM((STEP, 8), jnp.float32),
            pltpu.VMEM((ZROWS, 8), jnp.float32),
        ],
        compiler_params=pltpu.CompilerParams(use_tc_tiling_on_sc=False),
    )


def _dot(a, b):
    return jnp.dot(a, b, preferred_element_type=jnp.float32)


def _edge_body(nr_ref, ns_ref, ed_ref, *rest):
    (w1a, w1b, b1, w2, b2, w3, b3, w4, b4, w5, b5,
     u1a, u1b, c1, u2, c2, u3, c3, u4, c4, u5, c5) = rest[:-1]
    out_ref = rest[-1]

    xr = nr_ref[:, :3]
    xs = ns_ref[:, :3]
    h = _selu(_dot(xr, w1a[...]) + _dot(xs, w1b[...]) + b1[...])
    h = _selu(_dot(h, w2[...]) + b2[...])
    h = _selu(_dot(h, w3[...]) + b3[...])
    h = _selu(_dot(h, w4[...]) + b4[...])
    hn = _dot(h, w5[...]) + b5[...]

    cst = _dot(hn, u1b[...]) + c1[...]
    he = ed_ref[...]
    for _ in range(10):
        t = _selu(_dot(he, u1a[...]) + cst)
        t = _selu(_dot(t, u2[...]) + c2[...])
        t = _selu(_dot(t, u3[...]) + c3[...])
        t = _selu(_dot(t, u4[...]) + c4[...])
        he = _dot(t, u5[...]) + c5[...]
    out_ref[...] = jnp.concatenate(
        [he, jnp.zeros((he.shape[0], 5), jnp.float32)], axis=1)


def _tc_edge(nr, ns, edp, weights):
    full = [pl.BlockSpec(w.shape, lambda i, nd=w.ndim: (0,) * nd)
            for w in weights]
    return pl.pallas_call(
        _edge_body,
        grid=(EP // EB,),
        in_specs=[
            pl.BlockSpec((EB, 8), lambda i: (i, 0)),
            pl.BlockSpec((EB, 8), lambda i: (i, 0)),
            pl.BlockSpec((EB, 3), lambda i: (i, 0)),
        ] + full,
        out_specs=pl.BlockSpec((EB, 8), lambda i: (i, 0)),
        out_shape=jax.ShapeDtypeStruct((EP, 8), jnp.float32),
    )(nr, ns, edp, *weights)


def _node_body(acc_ref, nd_ref, *rest):
    (v1, d1, v2, d2, v3, d3, v4, d4, v5, d5,
     p1a, p1b, e1, p2, e2, p3, e3, p4, e4, p5, e5,
     q1, f1, q2, f2, q3, f3, q4, f4, q5, f5) = rest[:-2]
    out_ref, sums = rest[-2], rest[-1]

    i = pl.program_id(0)
    x = acc_ref[0] + acc_ref[1]
    e3v = x[:, :3]
    g = _selu(_dot(e3v, v1[...]) + d1[...])
    g = _selu(_dot(g, v2[...]) + d2[...])
    g = _selu(_dot(g, v3[...]) + d3[...])
    g = _selu(_dot(g, v4[...]) + d4[...])
    h2e = _dot(g, v5[...]) + d5[...]

    cst = _dot(h2e, p1b[...]) + e1[...]
    hn = nd_ref[:, :3]
    for _ in range(10):
        t = _selu(_dot(hn, p1a[...]) + cst)
        t = _selu(_dot(t, p2[...]) + e2[...])
        t = _selu(_dot(t, p3[...]) + e3[...])
        t = _selu(_dot(t, p4[...]) + e4[...])
        hn = _dot(t, p5[...]) + e5[...]

    rows = i * NB + lax.broadcasted_iota(jnp.int32, (NB, 1), 0)
    mask = rows < N_NODES
    sn = jnp.sum(jnp.where(mask, hn, 0.0), axis=0, keepdims=True)
    se = jnp.sum(jnp.where(mask, h2e, 0.0), axis=0, keepdims=True)
    part = jnp.concatenate([sn, se], axis=1)

    @pl.when(i == 0)
    def _():
        sums[...] = jnp.zeros_like(sums)

    sums[...] += part

    @pl.when(i == pl.num_programs(0) - 1)
    def _():
        t3 = sums[...]
        o = _selu(_dot(t3, q1[...]) + f1[...])
        o = _selu(_dot(o, q2[...]) + f2[...])
        o = _selu(_dot(o, q3[...]) + f3[...])
        o = _selu(_dot(o, q4[...]) + f4[...])
        out_ref[...] = _dot(o, q5[...]) + f5[...]


def _tc_node(acc, ndp, weights):
    full = [pl.BlockSpec(w.shape, lambda i, nd=w.ndim: (0,) * nd)
            for w in weights]
    return pl.pallas_call(
        _node_body,
        grid=(NPAD // NB,),
        in_specs=[
            pl.BlockSpec((SC_CORES, NB, 8), lambda i: (0, i, 0)),
            pl.BlockSpec((NB, 8), lambda i: (i, 0)),
        ] + full,
        out_specs=pl.BlockSpec((1, 9), lambda i: (0, 0)),
        out_shape=jax.ShapeDtypeStruct((1, 9), jnp.float32),
        scratch_shapes=[pltpu.VMEM((1, 19), jnp.float32)],
    )(acc, ndp, *weights)


def _flatten_mlp(mlp, split_first=None):
    out = []
    for k, (w, b) in enumerate(mlp):
        if k == 0 and split_first is not None:
            out.append(w[:split_first])
            out.append(w[split_first:])
        else:
            out.append(w)
        out.append(b.reshape(1, -1))
    return out


def kernel(nodes, edges, senders, receivers, params):
    f32 = jnp.float32
    nodes_tab = jnp.pad(nodes, ((0, 0), (0, 5)))
    recv_p = jnp.pad(receivers, (0, EP - N_EDGES))
    send_p = jnp.pad(senders, (0, EP - N_EDGES))
    recv2 = recv_p.reshape(EROWS, CHUNK)
    send2 = send_p.reshape(EROWS, CHUNK)

    nr, ns = _sc_gather()(nodes_tab, recv2, send2)

    ew = _flatten_mlp(params['permutate_nodes'], split_first=3) + \
        _flatten_mlp(params['update_edges'], split_first=3)
    edp = jnp.pad(edges, ((0, EP - N_EDGES), (0, 0)))
    h1 = _tc_edge(nr, ns, edp, ew)

    recv_s = jnp.pad(receivers, (0, EP - N_EDGES),
                     constant_values=N_NODES).reshape(EROWS, CHUNK)
    zrows = jnp.zeros((ZROWS, 8), f32)
    acc = _sc_scatter()(h1, recv_s, zrows)

    nw = _flatten_mlp(params['permutate_edges']) + \
        _flatten_mlp(params['update_nodes'], split_first=3) + \
        _flatten_mlp(params['predict'])
    ndp = jnp.pad(nodes, ((0, NPAD - N_NODES), (0, 5)))
    return _tc_node(acc, ndp, nw)
